# baseline (speedup 1.0000x reference)
import functools

import jax
import jax.numpy as jnp
from jax import lax
from jax.experimental import pallas as pl
from jax.experimental.pallas import tpu as pltpu
from jax.experimental.pallas import tpu_sc as plsc

_LANES = 16
_CHUNK = 64
_GCH = 4
_GROUP = _GCH + 1
_IRING = 3


def _gather_fn(n_total, n_vocab, d_model, n_cores, n_subcores, n_groups):
    n_workers = n_cores * n_subcores
    n_per_w = n_total // n_workers
    g_idx = _GROUP * _CHUNK
    vecs_per_row = d_model // _LANES

    mesh = plsc.VectorSubcoreMesh(core_axis_name="c", subcore_axis_name="s")

    @functools.partial(
        pl.kernel,
        out_type=jax.ShapeDtypeStruct((n_total, d_model), jnp.float32),
        mesh=mesh,
        scratch_types=[
            pltpu.VMEM_SHARED((n_vocab, d_model), jnp.float32),
            pltpu.VMEM((n_vocab, d_model), jnp.float32),
            pltpu.VMEM((_IRING * g_idx,), jnp.int32),
            pltpu.VMEM((2 * _GCH, _CHUNK, d_model), jnp.float32),
            pltpu.VMEM((_CHUNK, d_model), jnp.float32),
            pltpu.SemaphoreType.DMA((2 * _GCH,)),
            pltpu.SemaphoreType.DMA((2 * _GCH,)),
            pltpu.SemaphoreType.DMA((1,)),
            pltpu.SemaphoreType.DMA((_IRING,)),
        ],
    )
    def run(idx_hbm, table_hbm, out_hbm, table_s, table_v,
            idx_ring, rows_v, build_v, sem_g, sem_s, sem_b, sem_i):
        sid = lax.axis_index("s")
        wid = sid * n_cores + lax.axis_index("c")
        base = wid * n_per_w

        @pl.when(sid == 0)
        def _():
            pltpu.sync_copy(table_hbm, table_s)

        pltpu.sync_copy(table_hbm, table_v)
        pltpu.sync_copy(idx_hbm.at[pl.ds(base, 2 * g_idx)],
                        idx_ring.at[pl.ds(0, 2 * g_idx)])
        plsc.subcore_barrier()

        def prefetch(g):
            islot = lax.rem(g, _IRING)
            return pltpu.make_async_copy(
                idx_hbm.at[pl.ds(base + g * g_idx, g_idx)],
                idx_ring.at[pl.ds(islot * g_idx, g_idx)],
                sem_i.at[islot],
            )

        def gather(g, b, slot):
            ioff = lax.rem(g, _IRING) * g_idx + b * _CHUNK
            return pltpu.make_async_copy(
                table_s.at[idx_ring.at[pl.ds(ioff, _CHUNK)]],
                rows_v.at[slot],
                sem_g.at[slot],
            )

        def gstore(g, b, slot):
            return pltpu.make_async_copy(
                rows_v.at[slot],
                out_hbm.at[pl.ds(base + (g * _GROUP + b) * _CHUNK, _CHUNK)],
                sem_s.at[slot],
            )

        def bstore(g):
            return pltpu.make_async_copy(
                build_v,
                out_hbm.at[pl.ds(base + (g * _GROUP + _GCH) * _CHUNK,
                                 _CHUNK)],
                sem_b.at[0],
            )

        def build(g):
            ibase = lax.rem(g, _IRING) * g_idx + _GCH * _CHUNK
            for q in range(_CHUNK // _LANES):
                ivec = idx_ring[pl.ds(ibase + q * _LANES, _LANES)]
                for u in range(_LANES):
                    src = ivec[u]
                    r = q * _LANES + u
                    for k in range(vecs_per_row):
                        build_v[r, pl.ds(k * _LANES, _LANES)] = (
                            table_v[src, pl.ds(k * _LANES, _LANES)])

        for b in range(_GCH):
            gather(0, b, b).start()

        def sub_body(g, bank):
            @pl.when(g + 2 < n_groups)
            def _():
                prefetch(g + 2).start()

            for b in range(_GCH):
                slot = bank * _GCH + b
                gather(g, b, slot).wait()
                gstore(g, b, slot).start()

            @pl.when(g + 1 < n_groups)
            def _():
                @pl.when(g >= 1)
                def _():
                    prefetch(g + 1).wait()
                for b in range(_GCH):
                    oslot = (1 - bank) * _GCH + b

                    @pl.when(g >= 1)
                    def _():
                        gstore(g - 1, b, oslot).wait()

                    gather(g + 1, b, oslot).start()

            @pl.when(g >= 1)
            def _():
                bstore(g - 1).wait()

            build(g)
            bstore(g).start()

        def body(t, carry):
            sub_body(2 * t, 0)
            sub_body(2 * t + 1, 1)
            return carry

        lax.fori_loop(0, n_groups // 2, body, 0)

        for b in range(2 * _GCH):
            gstore(n_groups - 2 + b // _GCH, b % _GCH, b).wait()
        bstore(n_groups - 1).wait()

    return run


def kernel(positions, pe):
    b, s = positions.shape
    v, d = pe.shape
    n_total = b * s
    idx_flat = positions.reshape(n_total).astype(jnp.int32)

    info = plsc.get_sparse_core_info()
    n_cores, n_subcores = info.num_cores, info.num_subcores
    n_workers = n_cores * n_subcores
    n_per_w = n_total // n_workers
    n_groups = n_per_w // (_GROUP * _CHUNK)

    out = _gather_fn(n_total, v, d, n_cores, n_subcores, n_groups)(
        idx_flat, pe
    )
    return out.reshape(b, s, d)


# speedup vs baseline: 1.5137x; 1.5137x over previous
import functools

import jax
import jax.numpy as jnp
from jax import lax
from jax.experimental import pallas as pl
from jax.experimental.pallas import tpu as pltpu
from jax.experimental.pallas import tpu_sc as plsc

_NSLOT = 5
_DEPTH = 4


def _gather_fn(n_total, n_vocab, d_model, n_cores, n_subcores, chunk,
               n_chunks):
    n_workers = n_cores * n_subcores
    n_per_w = n_total // n_workers

    mesh = plsc.VectorSubcoreMesh(core_axis_name="c", subcore_axis_name="s")

    @functools.partial(
        pl.kernel,
        out_type=jax.ShapeDtypeStruct((n_total, d_model), jnp.float32),
        mesh=mesh,
        scratch_types=[
            pltpu.VMEM_SHARED((n_vocab, d_model), jnp.float32),
            pltpu.VMEM((n_per_w,), jnp.int32),
            pltpu.VMEM((_NSLOT, chunk, d_model), jnp.float32),
            pltpu.SemaphoreType.DMA((_NSLOT,)),
            pltpu.SemaphoreType.DMA((_NSLOT,)),
        ],
    )
    def run(idx_hbm, table_hbm, out_hbm, table_s, idx_v, rows_v, sem_g,
            sem_s):
        sid = lax.axis_index("s")
        wid = sid * n_cores + lax.axis_index("c")
        base = wid * n_per_w

        @pl.when(sid == 0)
        def _():
            pltpu.sync_copy(table_hbm, table_s)

        pltpu.sync_copy(idx_hbm.at[pl.ds(base, n_per_w)], idx_v)
        plsc.subcore_barrier()

        def gather(i, slot):
            return pltpu.make_async_copy(
                table_s.at[idx_v.at[pl.ds(i * chunk, chunk)]],
                rows_v.at[slot],
                sem_g.at[slot],
            )

        def store(i, slot):
            return pltpu.make_async_copy(
                rows_v.at[slot],
                out_hbm.at[pl.ds(base + i * chunk, chunk)],
                sem_s.at[slot],
            )

        for b in range(_DEPTH):
            gather(b, b).start()

        for b in range(_NSLOT):
            gather(b, b).wait()
            store(b, b).start()
            nslot = (b + _DEPTH) % _NSLOT
            if b >= 1:
                store(b - 1, nslot).wait()
            gather(b + _DEPTH, nslot).start()

        def body(g, carry):
            for b in range(_NSLOT):
                i = g * _NSLOT + b
                nslot = (b + _DEPTH) % _NSLOT
                gather(i, b).wait()
                store(i, b).start()
                store(i - 1, nslot).wait()
                gather(i + _DEPTH, nslot).start()
            return carry

        lax.fori_loop(1, n_chunks // _NSLOT - 1, body, 0)

        g_last = n_chunks // _NSLOT - 1
        for b in range(_NSLOT):
            i = g_last * _NSLOT + b
            nslot = (b + _DEPTH) % _NSLOT
            gather(i, b).wait()
            store(i, b).start()
            if i + _DEPTH < n_chunks:
                store(i - 1, nslot).wait()
                gather(i + _DEPTH, nslot).start()

        for b in range(_NSLOT):
            store(g_last * _NSLOT + b, b).wait()

    return run


def kernel(positions, pe):
    b, s = positions.shape
    v, d = pe.shape
    n_total = b * s
    idx_flat = positions.reshape(n_total).astype(jnp.int32)

    info = plsc.get_sparse_core_info()
    n_cores, n_subcores = info.num_cores, info.num_subcores
    n_workers = n_cores * n_subcores
    n_per_w = n_total // n_workers
    chunk = 128
    n_chunks = n_per_w // chunk

    out = _gather_fn(n_total, v, d, n_cores, n_subcores, chunk, n_chunks)(
        idx_flat, pe
    )
    return out.reshape(b, s, d)
